# baseline (speedup 1.0000x reference)
import functools

import jax
import jax.numpy as jnp
from jax.experimental import pallas as pl
from jax.experimental.pallas import tpu as pltpu
from jax.experimental.pallas import tpu_sc as plsc

NC = 2
NS = 16
NW = NC * NS
CH = 128


def _sc_gather(node_emb, rel_emb, head_index, rel_type, tail_index):
    B = head_index.shape[0]
    D = node_emb.shape[1]
    bpw = B // NW
    nch = bpw // CH
    mesh = plsc.VectorSubcoreMesh(core_axis_name="c", subcore_axis_name="s")
    row_t = jax.ShapeDtypeStruct((B, D), jnp.float32)

    @functools.partial(
        pl.kernel,
        out_type=[row_t, row_t, row_t],
        mesh=mesh,
        compiler_params=pltpu.CompilerParams(use_tc_tiling_on_sc=False),
        scratch_types=[
            pltpu.VMEM((nch, CH), jnp.int32),
            pltpu.VMEM((nch, CH), jnp.int32),
            pltpu.VMEM((nch, CH), jnp.int32),
            pltpu.VMEM((bpw, D), jnp.float32),
            pltpu.VMEM((bpw, D), jnp.float32),
            pltpu.VMEM((bpw, D), jnp.float32),
            pltpu.SemaphoreType.DMA,
        ],
    )
    def gather_kernel(node_hbm, rel_hbm, hi_hbm, ri_hbm, ti_hbm,
                      h_out, r_out, t_out,
                      hi_v, ri_v, ti_v, h_v, r_v, t_v, sem):
        wid = jax.lax.axis_index("s") * NC + jax.lax.axis_index("c")
        base = wid * bpw
        for c in range(nch):
            off = base + c * CH
            pltpu.sync_copy(hi_hbm.at[pl.ds(off, CH)], hi_v.at[c])
            pltpu.sync_copy(ri_hbm.at[pl.ds(off, CH)], ri_v.at[c])
            pltpu.sync_copy(ti_hbm.at[pl.ds(off, CH)], ti_v.at[c])
        copies = []
        for c in range(nch):
            dst = pl.ds(c * CH, CH)
            copies.append(pltpu.async_copy(node_hbm.at[hi_v.at[c]], h_v.at[dst], sem))
            copies.append(pltpu.async_copy(rel_hbm.at[ri_v.at[c]], r_v.at[dst], sem))
            copies.append(pltpu.async_copy(node_hbm.at[ti_v.at[c]], t_v.at[dst], sem))
        for cp in copies:
            cp.wait()
        pltpu.sync_copy(h_v, h_out.at[pl.ds(base, bpw)])
        pltpu.sync_copy(r_v, r_out.at[pl.ds(base, bpw)])
        pltpu.sync_copy(t_v, t_out.at[pl.ds(base, bpw)])

    return gather_kernel(node_emb, rel_emb, head_index, rel_type, tail_index)


def _tc_body(h_ref, r_ref, t_ref, o_ref):
    h = h_ref[...]
    r = r_ref[...]
    t = t_ref[...]
    hn = jnp.sqrt(jnp.sum(h * h, axis=1, keepdims=True))
    tn = jnp.sqrt(jnp.sum(t * t, axis=1, keepdims=True))
    hu = h / jnp.maximum(hn, 1e-12)
    tu = t / jnp.maximum(tn, 1e-12)
    d = hu + r - tu
    o_ref[...] = -jnp.sqrt(jnp.sum(d * d, axis=1))


def _tc_score(h_rows, r_rows, t_rows):
    B, D = h_rows.shape
    blk = 2048
    grid = (B // blk,)
    spec = pl.BlockSpec((blk, D), lambda i: (i, 0))
    return pl.pallas_call(
        _tc_body,
        grid=grid,
        in_specs=[spec, spec, spec],
        out_specs=pl.BlockSpec((blk,), lambda i: (i,)),
        out_shape=jax.ShapeDtypeStruct((B,), jnp.float32),
    )(h_rows, r_rows, t_rows)


def kernel(head_index, rel_type, tail_index, node_emb, rel_emb):
    hi = head_index.astype(jnp.int32)
    ri = rel_type.astype(jnp.int32)
    ti = tail_index.astype(jnp.int32)
    h_rows, r_rows, t_rows = _sc_gather(node_emb, rel_emb, hi, ri, ti)
    return _tc_score(h_rows, r_rows, t_rows)


# speedup vs baseline: 1.6771x; 1.6771x over previous
import functools

import jax
from jax import lax
import jax.numpy as jnp
from jax.experimental import pallas as pl
from jax.experimental.pallas import tpu as pltpu
from jax.experimental.pallas import tpu_sc as plsc

NC = 2
NS = 16
NW = NC * NS
CH = 128


def _sc_gather(node_emb, rel_emb, head_index, rel_type, tail_index):
    B = head_index.shape[0]
    D = node_emb.shape[1]
    bpw = B // NW
    nch = bpw // CH
    row_bytes = D * 4
    mesh = plsc.VectorSubcoreMesh(core_axis_name="c", subcore_axis_name="s")
    row_t = jax.ShapeDtypeStruct((B, D), jnp.float32)

    @functools.partial(
        pl.kernel,
        out_type=[row_t, row_t, row_t],
        mesh=mesh,
        scratch_types=[
            pltpu.VMEM((3, CH), jnp.int32),
            pltpu.VMEM((2, CH, D), jnp.float32),
            pltpu.VMEM((2, CH, D), jnp.float32),
            pltpu.VMEM((2, CH, D), jnp.float32),
            pltpu.SemaphoreType.DMA,
            pltpu.SemaphoreType.DMA,
            pltpu.SemaphoreType.DMA,
        ],
    )
    def gather_kernel(node_hbm, rel_hbm, hi_hbm, ri_hbm, ti_hbm,
                      h_out, r_out, t_out,
                      idx_s, h_v, r_v, t_v, gsem, wsem0, wsem1):
        wid = lax.axis_index("s") * NC + lax.axis_index("c")
        base = wid * bpw
        wsems = (wsem0, wsem1)
        for c in range(nch):
            p = c % 2
            off = base + c * CH
            pltpu.sync_copy(hi_hbm.at[pl.ds(off, CH)], idx_s.at[0])
            pltpu.sync_copy(ri_hbm.at[pl.ds(off, CH)], idx_s.at[1])
            pltpu.sync_copy(ti_hbm.at[pl.ds(off, CH)], idx_s.at[2])
            if c >= 2:
                for buf, out in ((h_v, h_out), (r_v, r_out), (t_v, t_out)):
                    pltpu.make_async_copy(out.at[pl.ds(0, CH)], buf.at[p],
                                          wsems[p]).wait()

            @pl.loop(0, CH // 16)
            def _(g):
                b16 = g * 16
                hv16 = idx_s[0, pl.ds(b16, 16)]
                rv16 = idx_s[1, pl.ds(b16, 16)]
                tv16 = idx_s[2, pl.ds(b16, 16)]
                for j in range(16):
                    pltpu.async_copy(node_hbm.at[hv16[j]],
                                     h_v.at[p, b16 + j], gsem)
                    pltpu.async_copy(rel_hbm.at[rv16[j]],
                                     r_v.at[p, b16 + j], gsem)
                    pltpu.async_copy(node_hbm.at[tv16[j]],
                                     t_v.at[p, b16 + j], gsem)

            for buf, out in ((h_v, h_out), (r_v, r_out), (t_v, t_out)):
                pltpu.make_async_copy(out.at[pl.ds(0, CH)], buf.at[p],
                                      gsem).wait()
            pltpu.async_copy(h_v.at[p], h_out.at[pl.ds(off, CH)], wsems[p])
            pltpu.async_copy(r_v.at[p], r_out.at[pl.ds(off, CH)], wsems[p])
            pltpu.async_copy(t_v.at[p], t_out.at[pl.ds(off, CH)], wsems[p])
        for p in range(2):
            for buf, out in ((h_v, h_out), (r_v, r_out), (t_v, t_out)):
                pltpu.make_async_copy(out.at[pl.ds(0, CH)], buf.at[p],
                                      wsems[p]).wait()

    return gather_kernel(node_emb, rel_emb, head_index, rel_type, tail_index)


def _tc_body(h_ref, r_ref, t_ref, o_ref):
    h = h_ref[...]
    r = r_ref[...]
    t = t_ref[...]
    hn = jnp.sqrt(jnp.sum(h * h, axis=1, keepdims=True))
    tn = jnp.sqrt(jnp.sum(t * t, axis=1, keepdims=True))
    hu = h / jnp.maximum(hn, 1e-12)
    tu = t / jnp.maximum(tn, 1e-12)
    d = hu + r - tu
    o_ref[...] = -jnp.sqrt(jnp.sum(d * d, axis=1))


def _tc_score(h_rows, r_rows, t_rows):
    B, D = h_rows.shape
    blk = 2048
    grid = (B // blk,)
    spec = pl.BlockSpec((blk, D), lambda i: (i, 0))
    return pl.pallas_call(
        _tc_body,
        grid=grid,
        in_specs=[spec, spec, spec],
        out_specs=pl.BlockSpec((blk,), lambda i: (i,)),
        out_shape=jax.ShapeDtypeStruct((B,), jnp.float32),
    )(h_rows, r_rows, t_rows)


def kernel(head_index, rel_type, tail_index, node_emb, rel_emb):
    hi = head_index.astype(jnp.int32)
    ri = rel_type.astype(jnp.int32)
    ti = tail_index.astype(jnp.int32)
    h_rows, r_rows, t_rows = _sc_gather(node_emb, rel_emb, hi, ri, ti)
    return _tc_score(h_rows, r_rows, t_rows)


# speedup vs baseline: 1.6818x; 1.0028x over previous
import functools

import jax
from jax import lax
import jax.numpy as jnp
from jax.experimental import pallas as pl
from jax.experimental.pallas import tpu as pltpu
from jax.experimental.pallas import tpu_sc as plsc

NC = 2
NS = 16
NW = NC * NS
CH = 128


def _sc_gather(node_emb, rel_emb, head_index, rel_type, tail_index):
    B = head_index.shape[0]
    D = node_emb.shape[1]
    bpw = B // NW
    nch = bpw // CH
    row_bytes = D * 4
    mesh = plsc.VectorSubcoreMesh(core_axis_name="c", subcore_axis_name="s")
    row_t = jax.ShapeDtypeStruct((B, D), jnp.float32)

    @functools.partial(
        pl.kernel,
        out_type=[row_t, row_t, row_t],
        mesh=mesh,
        compiler_params=pltpu.CompilerParams(use_tc_tiling_on_sc=True),
        scratch_types=[
            pltpu.VMEM((3, CH), jnp.int32),
            pltpu.VMEM((2, CH, D), jnp.float32),
            pltpu.VMEM((2, CH, D), jnp.float32),
            pltpu.VMEM((2, CH, D), jnp.float32),
            pltpu.SemaphoreType.DMA,
            pltpu.SemaphoreType.DMA,
            pltpu.SemaphoreType.DMA,
        ],
    )
    def gather_kernel(node_hbm, rel_hbm, hi_hbm, ri_hbm, ti_hbm,
                      h_out, r_out, t_out,
                      idx_s, h_v, r_v, t_v, gsem, wsem0, wsem1):
        wid = lax.axis_index("s") * NC + lax.axis_index("c")
        base = wid * bpw
        wsems = (wsem0, wsem1)
        for c in range(nch):
            p = c % 2
            off = base + c * CH
            pltpu.sync_copy(hi_hbm.at[pl.ds(off, CH)], idx_s.at[0])
            pltpu.sync_copy(ri_hbm.at[pl.ds(off, CH)], idx_s.at[1])
            pltpu.sync_copy(ti_hbm.at[pl.ds(off, CH)], idx_s.at[2])
            if c >= 2:
                for buf, out in ((h_v, h_out), (r_v, r_out), (t_v, t_out)):
                    pltpu.make_async_copy(out.at[pl.ds(0, CH)], buf.at[p],
                                          wsems[p]).wait()

            @pl.loop(0, CH // 16)
            def _(g):
                b16 = g * 16
                hv16 = idx_s[0, pl.ds(b16, 16)]
                rv16 = idx_s[1, pl.ds(b16, 16)]
                tv16 = idx_s[2, pl.ds(b16, 16)]
                for j in range(16):
                    pltpu.async_copy(node_hbm.at[hv16[j]],
                                     h_v.at[p, b16 + j], gsem)
                    pltpu.async_copy(rel_hbm.at[rv16[j]],
                                     r_v.at[p, b16 + j], gsem)
                    pltpu.async_copy(node_hbm.at[tv16[j]],
                                     t_v.at[p, b16 + j], gsem)

            for buf, out in ((h_v, h_out), (r_v, r_out), (t_v, t_out)):
                pltpu.make_async_copy(out.at[pl.ds(0, CH)], buf.at[p],
                                      gsem).wait()
            pltpu.async_copy(h_v.at[p], h_out.at[pl.ds(off, CH)], wsems[p])
            pltpu.async_copy(r_v.at[p], r_out.at[pl.ds(off, CH)], wsems[p])
            pltpu.async_copy(t_v.at[p], t_out.at[pl.ds(off, CH)], wsems[p])
        for p in range(2):
            for buf, out in ((h_v, h_out), (r_v, r_out), (t_v, t_out)):
                pltpu.make_async_copy(out.at[pl.ds(0, CH)], buf.at[p],
                                      wsems[p]).wait()

    return gather_kernel(node_emb, rel_emb, head_index, rel_type, tail_index)


def _tc_body(h_ref, r_ref, t_ref, o_ref):
    h = h_ref[...]
    r = r_ref[...]
    t = t_ref[...]
    hn = jnp.sqrt(jnp.sum(h * h, axis=1, keepdims=True))
    tn = jnp.sqrt(jnp.sum(t * t, axis=1, keepdims=True))
    hu = h / jnp.maximum(hn, 1e-12)
    tu = t / jnp.maximum(tn, 1e-12)
    d = hu + r - tu
    o_ref[...] = -jnp.sqrt(jnp.sum(d * d, axis=1))


def _tc_score(h_rows, r_rows, t_rows):
    B, D = h_rows.shape
    blk = 2048
    grid = (B // blk,)
    spec = pl.BlockSpec((blk, D), lambda i: (i, 0))
    return pl.pallas_call(
        _tc_body,
        grid=grid,
        in_specs=[spec, spec, spec],
        out_specs=pl.BlockSpec((blk,), lambda i: (i,)),
        out_shape=jax.ShapeDtypeStruct((B,), jnp.float32),
    )(h_rows, r_rows, t_rows)


def kernel(head_index, rel_type, tail_index, node_emb, rel_emb):
    hi = head_index.astype(jnp.int32)
    ri = rel_type.astype(jnp.int32)
    ti = tail_index.astype(jnp.int32)
    h_rows, r_rows, t_rows = _sc_gather(node_emb, rel_emb, hi, ri, ti)
    return _tc_score(h_rows, r_rows, t_rows)
